# baseline (speedup 1.0000x reference)
import functools

import numpy as np

import jax
import jax.numpy as jnp
from jax.experimental import pallas as pl
from jax.experimental.pallas import tpu as pltpu


def _umerge_nhwc_kernel(over_ref, up_ref, w_ref, b_ref, q_ref, out_ref, *,
                        c_over, c_out, h, w):
    x = up_ref[0]
    y = jax.lax.dot_general(x, w_ref[...], (((1,), (0,)), ((), ())),
                            preferred_element_type=jnp.float32)
    y = (y + b_ref[...]).astype(out_ref.dtype)
    out_ref[0, :, :c_over] = over_ref[0]
    q = q_ref[...]
    for i in range(h):
        ys = jnp.concatenate(
            [y[i * w:(i + 1) * w, d * c_out:(d + 1) * c_out] for d in range(4)],
            axis=0)
        wi = jax.lax.dot_general(q, ys, (((1,), (0,)), ((), ())),
                                 preferred_element_type=jnp.float32)
        out_ref[0, i * 4 * w:(i + 1) * 4 * w, c_over:] = wi.astype(out_ref.dtype)


def kernel(over_nchw, up_nchw, weight, bias):
    B, c_in, h, w = up_nchw.shape
    c_out = weight.shape[1]
    c_over, Ho, Wo = over_nchw.shape[1], over_nchw.shape[2], over_nchw.shape[3]
    H, W = 2 * h, 2 * w
    c_total = c_over + c_out
    out_dtype = up_nchw.dtype

    if Ho != H:
        bh = (Ho - H) // 2
        over_nchw = over_nchw[:, :, bh:bh + H, :]
    if Wo != W:
        bw = (Wo - W) // 2
        over_nchw = over_nchw[:, :, :, bw:bw + W]

    up_r = jnp.transpose(up_nchw, (0, 2, 3, 1)).reshape(B, h * w, c_in)
    over_r = jnp.transpose(over_nchw, (0, 2, 3, 1)).reshape(B, H * W, c_over)

    wmat = jnp.transpose(weight, (0, 2, 3, 1)).reshape(c_in, 4 * c_out)
    b2 = jnp.tile(bias, 4).reshape(1, 4 * c_out).astype(jnp.float32)

    q = np.zeros((4 * w, 4 * w), np.float32)
    r = np.arange(4 * w)
    di, dj, j = (r // w) // 2, (r // w) % 2, r % w
    q[di * W + 2 * j + dj, r] = 1.0
    qmat = jnp.asarray(q)

    itemsize = jnp.dtype(out_dtype).itemsize
    cost = pl.CostEstimate(
        flops=2 * B * h * w * c_in * 4 * c_out + 2 * B * h * 4 * w * 4 * w * c_out,
        transcendentals=0,
        bytes_accessed=(up_r.size + over_r.size + B * H * W * c_total) * itemsize,
    )

    T = 4 if h % 4 == 0 else (2 if h % 2 == 0 else 1)
    hs = h // T
    body = functools.partial(_umerge_nhwc_kernel, c_over=c_over, c_out=c_out,
                             h=hs, w=w)
    out_r = pl.pallas_call(
        body,
        out_shape=jax.ShapeDtypeStruct((B, H * W, c_total), out_dtype),
        grid=(B, T),
        in_specs=[
            pl.BlockSpec((1, H * W // T, c_over), lambda b, t: (b, t, 0)),
            pl.BlockSpec((1, h * w // T, c_in), lambda b, t: (b, t, 0)),
            pl.BlockSpec((c_in, 4 * c_out), lambda b, t: (0, 0)),
            pl.BlockSpec((1, 4 * c_out), lambda b, t: (0, 0)),
            pl.BlockSpec((4 * w, 4 * w), lambda b, t: (0, 0)),
        ],
        out_specs=pl.BlockSpec((1, H * W // T, c_total), lambda b, t: (b, t, 0)),
        compiler_params=pltpu.CompilerParams(
            dimension_semantics=("parallel", "parallel")),
        cost_estimate=cost,
    )(over_r, up_r, wmat, b2, qmat)

    return jnp.transpose(out_r.reshape(B, H, W, c_total), (0, 3, 1, 2))


# speedup vs baseline: 1.3505x; 1.3505x over previous
import functools

import numpy as np

import jax
import jax.numpy as jnp
from jax.experimental import pallas as pl
from jax.experimental.pallas import tpu as pltpu


def _umerge_nhwc_kernel(over_ref, up_ref, w_ref, b_ref, q_ref, out_ref, *,
                        c_over, c_out, h, w):
    x = up_ref[0]
    y = jax.lax.dot_general(x, w_ref[...], (((1,), (0,)), ((), ())),
                            preferred_element_type=jnp.float32)
    y = (y + b_ref[...]).astype(out_ref.dtype)
    out_ref[0, :, :c_over] = over_ref[0]
    q = q_ref[...]
    for i in range(h):
        ys = jnp.concatenate(
            [y[i * w:(i + 1) * w, d * c_out:(d + 1) * c_out] for d in range(4)],
            axis=0)
        wi = jax.lax.dot_general(q, ys, (((1,), (0,)), ((), ())),
                                 preferred_element_type=jnp.float32)
        out_ref[0, i * 4 * w:(i + 1) * 4 * w, c_over:] = wi.astype(out_ref.dtype)


def kernel(over_nchw, up_nchw, weight, bias):
    B, c_in, h, w = up_nchw.shape
    c_out = weight.shape[1]
    c_over, Ho, Wo = over_nchw.shape[1], over_nchw.shape[2], over_nchw.shape[3]
    H, W = 2 * h, 2 * w
    c_total = c_over + c_out
    out_dtype = up_nchw.dtype

    if Ho != H:
        bh = (Ho - H) // 2
        over_nchw = over_nchw[:, :, bh:bh + H, :]
    if Wo != W:
        bw = (Wo - W) // 2
        over_nchw = over_nchw[:, :, :, bw:bw + W]

    up_r = jnp.transpose(up_nchw, (0, 2, 3, 1)).reshape(B, h * w, c_in)
    over_r = jnp.transpose(over_nchw, (0, 2, 3, 1)).reshape(B, H * W, c_over)

    wmat = jnp.transpose(weight, (0, 2, 3, 1)).reshape(c_in, 4 * c_out)
    b2 = jnp.tile(bias, 4).reshape(1, 4 * c_out).astype(jnp.float32)

    q = np.zeros((4 * w, 4 * w), np.float32)
    r = np.arange(4 * w)
    di, dj, j = (r // w) // 2, (r // w) % 2, r % w
    q[di * W + 2 * j + dj, r] = 1.0
    qmat = jnp.asarray(q)

    itemsize = jnp.dtype(out_dtype).itemsize
    cost = pl.CostEstimate(
        flops=2 * B * h * w * c_in * 4 * c_out + 2 * B * h * 4 * w * 4 * w * c_out,
        transcendentals=0,
        bytes_accessed=(up_r.size + over_r.size + B * H * W * c_total) * itemsize,
    )

    T = 2 if h % 2 == 0 else 1
    hs = h // T
    body = functools.partial(_umerge_nhwc_kernel, c_over=c_over, c_out=c_out,
                             h=hs, w=w)
    out_r = pl.pallas_call(
        body,
        out_shape=jax.ShapeDtypeStruct((B, H * W, c_total), out_dtype),
        grid=(B, T),
        in_specs=[
            pl.BlockSpec((1, H * W // T, c_over), lambda b, t: (b, t, 0)),
            pl.BlockSpec((1, h * w // T, c_in), lambda b, t: (b, t, 0)),
            pl.BlockSpec((c_in, 4 * c_out), lambda b, t: (0, 0)),
            pl.BlockSpec((1, 4 * c_out), lambda b, t: (0, 0)),
            pl.BlockSpec((4 * w, 4 * w), lambda b, t: (0, 0)),
        ],
        out_specs=pl.BlockSpec((1, H * W // T, c_total), lambda b, t: (b, t, 0)),
        compiler_params=pltpu.CompilerParams(
            dimension_semantics=("parallel", "parallel")),
        cost_estimate=cost,
    )(over_r, up_r, wmat, b2, qmat)

    return jnp.transpose(out_r.reshape(B, H, W, c_total), (0, 3, 1, 2))


# speedup vs baseline: 1.6440x; 1.2173x over previous
import functools

import numpy as np

import jax
import jax.numpy as jnp
from jax.experimental import pallas as pl
from jax.experimental.pallas import tpu as pltpu


def _umerge_nhwc_kernel(over_ref, up_ref, w_ref, b_ref, q_ref, out_ref, *,
                        c_over, c_out, h, w):
    q = q_ref[...]
    for bb in range(up_ref.shape[0]):
        x = up_ref[bb]
        y = jax.lax.dot_general(x, w_ref[...], (((1,), (0,)), ((), ())),
                                preferred_element_type=jnp.float32)
        y = (y + b_ref[...]).astype(out_ref.dtype)
        out_ref[bb, :, :c_over] = over_ref[bb]
        for i in range(h):
            ys = jnp.concatenate(
                [y[i * w:(i + 1) * w, d * c_out:(d + 1) * c_out]
                 for d in range(4)],
                axis=0)
            wi = jax.lax.dot_general(q, ys, (((1,), (0,)), ((), ())),
                                     preferred_element_type=jnp.float32)
            out_ref[bb, i * 4 * w:(i + 1) * 4 * w, c_over:] = (
                wi.astype(out_ref.dtype))


def kernel(over_nchw, up_nchw, weight, bias):
    B, c_in, h, w = up_nchw.shape
    c_out = weight.shape[1]
    c_over, Ho, Wo = over_nchw.shape[1], over_nchw.shape[2], over_nchw.shape[3]
    H, W = 2 * h, 2 * w
    c_total = c_over + c_out
    out_dtype = up_nchw.dtype

    if Ho != H:
        bh = (Ho - H) // 2
        over_nchw = over_nchw[:, :, bh:bh + H, :]
    if Wo != W:
        bw = (Wo - W) // 2
        over_nchw = over_nchw[:, :, :, bw:bw + W]

    up_r = jnp.transpose(up_nchw, (0, 2, 3, 1)).reshape(B, h * w, c_in)
    over_r = jnp.transpose(over_nchw, (0, 2, 3, 1)).reshape(B, H * W, c_over)

    wmat = jnp.transpose(weight, (0, 2, 3, 1)).reshape(c_in, 4 * c_out)
    b2 = jnp.tile(bias, 4).reshape(1, 4 * c_out).astype(jnp.float32)

    q = np.zeros((4 * w, 4 * w), np.float32)
    r = np.arange(4 * w)
    di, dj, j = (r // w) // 2, (r // w) % 2, r % w
    q[di * W + 2 * j + dj, r] = 1.0
    qmat = jnp.asarray(q)

    itemsize = jnp.dtype(out_dtype).itemsize
    cost = pl.CostEstimate(
        flops=2 * B * h * w * c_in * 4 * c_out + 2 * B * h * 4 * w * 4 * w * c_out,
        transcendentals=0,
        bytes_accessed=(up_r.size + over_r.size + B * H * W * c_total) * itemsize,
    )

    BB = 2 if B % 2 == 0 else 1
    body = functools.partial(_umerge_nhwc_kernel, c_over=c_over, c_out=c_out,
                             h=h, w=w)
    out_r = pl.pallas_call(
        body,
        out_shape=jax.ShapeDtypeStruct((B, H * W, c_total), out_dtype),
        grid=(B // BB,),
        in_specs=[
            pl.BlockSpec((BB, H * W, c_over), lambda b: (b, 0, 0)),
            pl.BlockSpec((BB, h * w, c_in), lambda b: (b, 0, 0)),
            pl.BlockSpec((c_in, 4 * c_out), lambda b: (0, 0)),
            pl.BlockSpec((1, 4 * c_out), lambda b: (0, 0)),
            pl.BlockSpec((4 * w, 4 * w), lambda b: (0, 0)),
        ],
        out_specs=pl.BlockSpec((BB, H * W, c_total), lambda b: (b, 0, 0)),
        compiler_params=pltpu.CompilerParams(
            dimension_semantics=("parallel",)),
        cost_estimate=cost,
    )(over_r, up_r, wmat, b2, qmat)

    return jnp.transpose(out_r.reshape(B, H, W, c_total), (0, 3, 1, 2))
